# baseline (device time: 738438 ns/iter reference)
import jax
import jax.numpy as jnp
from jax import lax
from jax.experimental import pallas as pl
from jax.experimental.pallas import tpu as pltpu

N_DEV = 16
_GELU_C = 0.7978845608028654


def _gelu(y):
    return 0.5 * y * (1.0 + jnp.tanh(_GELU_C * (y + 0.044715 * y * y * y)))


def kernel(x, w_mat):
    m_per, k = x.shape
    _, n_per = w_mat.shape

    def body(x_ref, w_ref, out_ref, comm_ref, send_sems, recv_sems):
        my_pos = lax.axis_index("i")
        left = (my_pos - 1) % N_DEV
        right = (my_pos + 1) % N_DEV

        barrier_sem = pltpu.get_barrier_semaphore()
        for nbr in [left, right]:
            pl.semaphore_signal(
                barrier_sem, inc=1,
                device_id=(nbr,), device_id_type=pl.DeviceIdType.MESH,
            )
        pl.semaphore_wait(barrier_sem, 2)

        comm_ref[0, :, :] = x_ref[:, :]
        y = jnp.dot(x_ref[:, :], w_ref[:, :], preferred_element_type=jnp.float32)
        out_ref[pl.ds(my_pos * m_per, m_per), :] = _gelu(y)

        for h in range(N_DEV - 1):
            send_slot = h % 2
            recv_slot = (h + 1) % 2
            rdma = pltpu.make_async_remote_copy(
                src_ref=comm_ref.at[send_slot],
                dst_ref=comm_ref.at[recv_slot],
                send_sem=send_sems.at[send_slot],
                recv_sem=recv_sems.at[recv_slot],
                device_id=(right,),
                device_id_type=pl.DeviceIdType.MESH,
            )
            rdma.start()
            rdma.wait()

            origin = (my_pos - h - 1) % N_DEV
            y = jnp.dot(
                comm_ref[recv_slot, :, :], w_ref[:, :],
                preferred_element_type=jnp.float32,
            )
            out_ref[pl.ds(origin * m_per, m_per), :] = _gelu(y)

    return pl.pallas_call(
        body,
        out_shape=jax.ShapeDtypeStruct((N_DEV * m_per, n_per), jnp.float32),
        in_specs=[
            pl.BlockSpec(memory_space=pltpu.VMEM),
            pl.BlockSpec(memory_space=pltpu.VMEM),
        ],
        out_specs=pl.BlockSpec(memory_space=pltpu.VMEM),
        scratch_shapes=[
            pltpu.VMEM((2, m_per, k), jnp.float32),
            pltpu.SemaphoreType.DMA((2,)),
            pltpu.SemaphoreType.DMA((2,)),
        ],
        compiler_params=pltpu.CompilerParams(collective_id=0),
    )(x, w_mat)


# device time: 397603 ns/iter; 1.8572x vs baseline; 1.8572x over previous
import jax
import jax.numpy as jnp
from jax import lax
from jax.experimental import pallas as pl
from jax.experimental.pallas import tpu as pltpu

N_DEV = 16
N_HOP = N_DEV - 1
S = 3
_GELU_C = 0.7978845608028654


def _gelu(y):
    return 0.5 * y * (1.0 + jnp.tanh(_GELU_C * (y + 0.044715 * y * y * y)))


def kernel(x, w_mat):
    m_per, k = x.shape
    _, n_per = w_mat.shape
    half = m_per // 2

    def body(x_ref, w_ref, out_ref,
             cw_buf, ccw_buf, cw_send, cw_recv, ccw_send, ccw_recv,
             cw_credit, ccw_credit):
        my_pos = lax.axis_index("i")
        left = (my_pos - 1) % N_DEV
        right = (my_pos + 1) % N_DEV

        def mk(h, buf, send_sems, recv_sems, dst):
            return pltpu.make_async_remote_copy(
                src_ref=buf.at[h % S],
                dst_ref=buf.at[(h + 1) % S],
                send_sem=send_sems.at[h % S],
                recv_sem=recv_sems.at[(h + 1) % S],
                device_id=(dst,),
                device_id_type=pl.DeviceIdType.MESH,
            )

        def mk_cw(h):
            return mk(h, cw_buf, cw_send, cw_recv, right)

        def mk_ccw(h):
            return mk(h, ccw_buf, ccw_send, ccw_recv, left)

        barrier_sem = pltpu.get_barrier_semaphore()
        for nbr in [left, right]:
            pl.semaphore_signal(
                barrier_sem, inc=1,
                device_id=(nbr,), device_id_type=pl.DeviceIdType.MESH,
            )
        pl.semaphore_wait(barrier_sem, 2)

        cw_buf[0, :, :] = x_ref[:half, :]
        ccw_buf[0, :, :] = x_ref[half:, :]

        for h in range(N_HOP):
            if h > 0:
                mk_cw(h - 1).wait_recv()
                mk_ccw(h - 1).wait_recv()
            if h >= S - 1:
                pl.semaphore_wait(cw_credit, 1)
                pl.semaphore_wait(ccw_credit, 1)
            send_cw = mk_cw(h)
            send_ccw = mk_ccw(h)
            send_cw.start()
            send_ccw.start()
            if h == 0:
                y = jnp.dot(x_ref[:, :], w_ref[:, :],
                            preferred_element_type=jnp.float32)
                out_ref[pl.ds(my_pos * m_per, m_per), :] = _gelu(y)
            else:
                org_a = (my_pos - h) % N_DEV
                y = jnp.dot(cw_buf[h % S, :, :], w_ref[:, :],
                            preferred_element_type=jnp.float32)
                out_ref[pl.ds(org_a * m_per, half), :] = _gelu(y)
                org_b = (my_pos + h) % N_DEV
                y = jnp.dot(ccw_buf[h % S, :, :], w_ref[:, :],
                            preferred_element_type=jnp.float32)
                out_ref[pl.ds(org_b * m_per + half, half), :] = _gelu(y)
            send_cw.wait_send()
            send_ccw.wait_send()
            if h <= N_HOP - S:
                pl.semaphore_signal(
                    cw_credit, inc=1,
                    device_id=(left,), device_id_type=pl.DeviceIdType.MESH,
                )
                pl.semaphore_signal(
                    ccw_credit, inc=1,
                    device_id=(right,), device_id_type=pl.DeviceIdType.MESH,
                )

        mk_cw(N_HOP - 1).wait_recv()
        mk_ccw(N_HOP - 1).wait_recv()
        org_a = (my_pos - N_HOP) % N_DEV
        y = jnp.dot(cw_buf[N_HOP % S, :, :], w_ref[:, :],
                    preferred_element_type=jnp.float32)
        out_ref[pl.ds(org_a * m_per, half), :] = _gelu(y)
        org_b = (my_pos + N_HOP) % N_DEV
        y = jnp.dot(ccw_buf[N_HOP % S, :, :], w_ref[:, :],
                    preferred_element_type=jnp.float32)
        out_ref[pl.ds(org_b * m_per + half, half), :] = _gelu(y)

    return pl.pallas_call(
        body,
        out_shape=jax.ShapeDtypeStruct((N_DEV * m_per, n_per), jnp.float32),
        in_specs=[
            pl.BlockSpec(memory_space=pltpu.VMEM),
            pl.BlockSpec(memory_space=pltpu.VMEM),
        ],
        out_specs=pl.BlockSpec(memory_space=pltpu.VMEM),
        scratch_shapes=[
            pltpu.VMEM((S, half, k), jnp.float32),
            pltpu.VMEM((S, half, k), jnp.float32),
            pltpu.SemaphoreType.DMA((S,)),
            pltpu.SemaphoreType.DMA((S,)),
            pltpu.SemaphoreType.DMA((S,)),
            pltpu.SemaphoreType.DMA((S,)),
            pltpu.SemaphoreType.REGULAR,
            pltpu.SemaphoreType.REGULAR,
        ],
        compiler_params=pltpu.CompilerParams(collective_id=0),
    )(x, w_mat)


# device time: 364869 ns/iter; 2.0238x vs baseline; 1.0897x over previous
import jax
import jax.numpy as jnp
from jax import lax
from jax.experimental import pallas as pl
from jax.experimental.pallas import tpu as pltpu

N_DEV = 16
N_HOP = N_DEV - 1
S = 3
N_RING = 4
_GELU_C = 0.7978845608028654


def _gelu(y):
    return 0.5 * y * (1.0 + jnp.tanh(_GELU_C * (y + 0.044715 * y * y * y)))


def kernel(x, w_mat):
    m_per, k = x.shape
    _, n_per = w_mat.shape
    q = m_per // N_RING

    def body(x_ref, w_ref, out_ref, *scratch):
        bufs = scratch[0:4]
        send_sems = scratch[4:8]
        recv_sems = scratch[8:12]
        credits = scratch[12:16]

        my_pos = lax.axis_index("i")
        left = (my_pos - 1) % N_DEV
        right = (my_pos + 1) % N_DEV
        cw = (right, left, -1)
        ccw = (left, right, +1)
        ring_cfg = [cw, cw, ccw, ccw]
        ring_order = [0, 2, 1, 3]

        def mk(r, h):
            dst, _, _ = ring_cfg[r]
            return pltpu.make_async_remote_copy(
                src_ref=bufs[r].at[h % S],
                dst_ref=bufs[r].at[(h + 1) % S],
                send_sem=send_sems[r].at[h % S],
                recv_sem=recv_sems[r].at[(h + 1) % S],
                device_id=(dst,),
                device_id_type=pl.DeviceIdType.MESH,
            )

        barrier_sem = pltpu.get_barrier_semaphore()
        for nbr in [left, right]:
            pl.semaphore_signal(
                barrier_sem, inc=1,
                device_id=(nbr,), device_id_type=pl.DeviceIdType.MESH,
            )
        pl.semaphore_wait(barrier_sem, 2)

        for r in range(N_RING):
            bufs[r][0, :, :] = x_ref[pl.ds(r * q, q), :]

        for h in range(N_HOP):
            for r in ring_order:
                if h > 0:
                    mk(r, h - 1).wait_recv()
                if h >= S - 1:
                    pl.semaphore_wait(credits[r], 1)
                mk(r, h).start()
            if h == 0:
                y = jnp.dot(x_ref[:, :], w_ref[:, :],
                            preferred_element_type=jnp.float32)
                out_ref[pl.ds(my_pos * m_per, m_per), :] = _gelu(y)
            else:
                for r in range(N_RING):
                    _, _, sign = ring_cfg[r]
                    origin = (my_pos + sign * h) % N_DEV
                    y = jnp.dot(bufs[r][h % S, :, :], w_ref[:, :],
                                preferred_element_type=jnp.float32)
                    out_ref[pl.ds(origin * m_per + r * q, q), :] = _gelu(y)
            for r in ring_order:
                mk(r, h).wait_send()
            if h <= N_HOP - S:
                for r in range(N_RING):
                    _, upstream, _ = ring_cfg[r]
                    pl.semaphore_signal(
                        credits[r], inc=1,
                        device_id=(upstream,),
                        device_id_type=pl.DeviceIdType.MESH,
                    )

        for r in ring_order:
            mk(r, N_HOP - 1).wait_recv()
        for r in range(N_RING):
            _, _, sign = ring_cfg[r]
            origin = (my_pos + sign * N_HOP) % N_DEV
            y = jnp.dot(bufs[r][N_HOP % S, :, :], w_ref[:, :],
                        preferred_element_type=jnp.float32)
            out_ref[pl.ds(origin * m_per + r * q, q), :] = _gelu(y)

    return pl.pallas_call(
        body,
        out_shape=jax.ShapeDtypeStruct((N_DEV * m_per, n_per), jnp.float32),
        in_specs=[
            pl.BlockSpec(memory_space=pltpu.VMEM),
            pl.BlockSpec(memory_space=pltpu.VMEM),
        ],
        out_specs=pl.BlockSpec(memory_space=pltpu.VMEM),
        scratch_shapes=(
            [pltpu.VMEM((S, q, k), jnp.float32)] * N_RING
            + [pltpu.SemaphoreType.DMA((S,))] * N_RING
            + [pltpu.SemaphoreType.DMA((S,))] * N_RING
            + [pltpu.SemaphoreType.REGULAR] * N_RING
        ),
        compiler_params=pltpu.CompilerParams(collective_id=0),
    )(x, w_mat)


# device time: 309037 ns/iter; 2.3895x vs baseline; 1.1807x over previous
import jax
import jax.numpy as jnp
from jax import lax
from jax.experimental import pallas as pl
from jax.experimental.pallas import tpu as pltpu

N_DEV = 16
_GELU_C = 0.7978845608028654

_SEND_UP = {
    0: [(0, 0, 1), (1, 0, 1), (2, 0, 1)],
    1: [(1, 1, 3), (2, 1, 2)],
    2: [(2, 3, 3)],
}
_SEND_DN = {
    0: [(1, 0, 1), (2, 0, 2), (3, 0, 2)],
    1: [(1, 2, 2), (2, 2, 3)],
    2: [(1, 3, 3)],
}
_RECV_UP = {0: [(1, 1), (2, 1), (3, 1)], 1: [(2, 3), (3, 2)], 2: [(3, 3)]}
_RECV_DN = {0: [(0, 1), (1, 2), (2, 2)], 1: [(0, 2), (1, 3)], 2: [(0, 3)]}


def _gelu(y):
    return 0.5 * y * (1.0 + jnp.tanh(_GELU_C * (y + 0.044715 * y * y * y)))


def kernel(x, w_mat):
    m_per, k_dim = x.shape
    _, n_per = w_mat.shape
    half = m_per // 2

    def body(x_ref, w_ref, out_ref,
             stgA, stgB, pcw, pccw,
             up_send, up_recv, dn_send, dn_recv,
             h0cw_send, h0ccw_send, pcw_send, pcw_recv, pccw_send, pccw_recv,
             cred_cw, cred_ccw, exit_sem):
        my = lax.axis_index("i")
        z = my // 4
        j = my % 4
        jr_id = 4 * z + (j + 1) % 4
        jl_id = 4 * z + (j - 1) % 4
        up_id = my + 4
        dn_id = my - 4

        def gemm_store(src_val, origin_id, row_off):
            y = jnp.dot(src_val, w_ref[:, :],
                        preferred_element_type=jnp.float32)
            out_ref[pl.ds(origin_id * m_per + row_off, half), :] = _gelu(y)

        def zdesc(hf, up, kk, src_slot, dst_slot, dev):
            buf = stgA if hf == 0 else stgB
            ssem = up_send if up else dn_send
            rsem = up_recv if up else dn_recv
            return pltpu.make_async_remote_copy(
                src_ref=buf.at[src_slot],
                dst_ref=buf.at[dst_slot],
                send_sem=ssem.at[2 * kk + hf],
                recv_sem=rsem.at[2 * kk + hf],
                device_id=(dev,),
                device_id_type=pl.DeviceIdType.MESH,
            )

        def pdesc(t, cw):
            h, r = t % 3, t // 3
            buf = pcw if cw else pccw
            stg = stgA if cw else stgB
            h0s = h0cw_send if cw else h0ccw_send
            psend = pcw_send if cw else pccw_send
            precv = pcw_recv if cw else pccw_recv
            dev = jr_id if cw else jl_id
            if h == 0:
                src, ssem = stg.at[r], h0s.at[r]
            else:
                src, ssem = buf.at[(t - 1) % 3], psend.at[t % 3]
            return pltpu.make_async_remote_copy(
                src_ref=src, dst_ref=buf.at[t % 3],
                send_sem=ssem, recv_sem=precv.at[t % 3],
                device_id=(dev,), device_id_type=pl.DeviceIdType.MESH,
            )

        def ozr(r):
            if r == 0:
                return z
            if r == 1:
                return jnp.where(z == 0, 1, z - 1)
            if r == 2:
                return jnp.where(z == 0, 2,
                                 jnp.where(z == 1, 2, jnp.where(z == 2, 3, 1)))
            return jnp.where(z <= 1, 3, 0)

        def z_sends(kk):
            for up, table, dev in ((True, _SEND_UP, up_id), (False, _SEND_DN, dn_id)):
                for zv, src_slot, dst_slot in table[kk]:
                    @pl.when(z == zv)
                    def _():
                        for hf in (0, 1):
                            zdesc(hf, up, kk, src_slot, dst_slot, dev).start()

        def z_block(kk):
            for up, rtable, dev in ((True, _RECV_UP, dn_id), (False, _RECV_DN, up_id)):
                for zv, slot in rtable[kk]:
                    @pl.when(z == zv)
                    def _():
                        for hf in (0, 1):
                            zdesc(hf, up, kk, 0, slot, dev).wait_recv()
            if kk + 1 in _SEND_UP or kk + 1 in _SEND_DN:
                z_sends(kk + 1)
            for up, rtable in ((True, _RECV_UP), (False, _RECV_DN)):
                for zv, slot in rtable[kk]:
                    oid = 4 * ((zv - 1 - kk) if up else (zv + 1 + kk)) + j
                    @pl.when(z == zv)
                    def _():
                        gemm_store(stgA[slot, :, :], oid, 0)
                        gemm_store(stgB[slot, :, :], oid, half)

        barrier_sem = pltpu.get_barrier_semaphore()
        for nbr in (jl_id, jr_id):
            pl.semaphore_signal(barrier_sem, inc=1, device_id=(nbr,),
                                device_id_type=pl.DeviceIdType.MESH)

        @pl.when(z < 3)
        def _():
            pl.semaphore_signal(barrier_sem, inc=1, device_id=(up_id,),
                                device_id_type=pl.DeviceIdType.MESH)

        @pl.when(z > 0)
        def _():
            pl.semaphore_signal(barrier_sem, inc=1, device_id=(dn_id,),
                                device_id_type=pl.DeviceIdType.MESH)

        pl.semaphore_wait(barrier_sem, 2)

        @pl.when(z < 3)
        def _():
            pl.semaphore_wait(barrier_sem, 1)

        @pl.when(z > 0)
        def _():
            pl.semaphore_wait(barrier_sem, 1)

        stgA[0, :, :] = x_ref[:half, :]
        stgB[0, :, :] = x_ref[half:, :]
        z_sends(0)

        for t in range(12):
            if t >= 1:
                pdesc(t - 1, True).wait_recv()
                pdesc(t - 1, False).wait_recv()
            if t >= 3:
                pl.semaphore_wait(cred_cw, 1)
                pl.semaphore_wait(cred_ccw, 1)
            dcw = pdesc(t, True)
            dccw = pdesc(t, False)
            dcw.start()
            dccw.start()
            if t == 0:
                y = jnp.dot(x_ref[:, :], w_ref[:, :],
                            preferred_element_type=jnp.float32)
                out_ref[pl.ds(my * m_per, m_per), :] = _gelu(y)
            else:
                tp, hp, rp = t - 1, (t - 1) % 3, (t - 1) // 3
                ozp = ozr(rp)
                gemm_store(pcw[tp % 3, :, :], 4 * ozp + (j - hp - 1) % 4, 0)
                gemm_store(pccw[tp % 3, :, :], 4 * ozp + (j + hp + 1) % 4, half)
            if t % 3 != 0:
                dcw.wait_send()
                dccw.wait_send()
            if 1 <= t <= 9:
                pl.semaphore_signal(cred_cw, inc=1, device_id=(jl_id,),
                                    device_id_type=pl.DeviceIdType.MESH)
                pl.semaphore_signal(cred_ccw, inc=1, device_id=(jr_id,),
                                    device_id_type=pl.DeviceIdType.MESH)
            if t in (2, 5, 8):
                z_block(t // 3)

        pdesc(11, True).wait_recv()
        pdesc(11, False).wait_recv()
        oz3 = ozr(3)
        gemm_store(pcw[11 % 3, :, :], 4 * oz3 + (j + 1) % 4, 0)
        gemm_store(pccw[11 % 3, :, :], 4 * oz3 + (j - 1) % 4, half)

        for r in range(4):
            pdesc(3 * r, True).wait_send()
            pdesc(3 * r, False).wait_send()
        for kk in range(3):
            for up, table, dev in ((True, _SEND_UP, up_id), (False, _SEND_DN, dn_id)):
                for zv, src_slot, dst_slot in table[kk]:
                    @pl.when(z == zv)
                    def _():
                        for hf in (0, 1):
                            zdesc(hf, up, kk, src_slot, dst_slot, dev).wait_send()

        for nbr in (jl_id, jr_id):
            pl.semaphore_signal(exit_sem, inc=1, device_id=(nbr,),
                                device_id_type=pl.DeviceIdType.MESH)

        @pl.when(z < 3)
        def _():
            pl.semaphore_signal(exit_sem, inc=1, device_id=(up_id,),
                                device_id_type=pl.DeviceIdType.MESH)

        @pl.when(z > 0)
        def _():
            pl.semaphore_signal(exit_sem, inc=1, device_id=(dn_id,),
                                device_id_type=pl.DeviceIdType.MESH)

        pl.semaphore_wait(exit_sem, 2)

        @pl.when(z < 3)
        def _():
            pl.semaphore_wait(exit_sem, 1)

        @pl.when(z > 0)
        def _():
            pl.semaphore_wait(exit_sem, 1)

    return pl.pallas_call(
        body,
        out_shape=jax.ShapeDtypeStruct((N_DEV * m_per, n_per), jnp.float32),
        in_specs=[
            pl.BlockSpec(memory_space=pltpu.VMEM),
            pl.BlockSpec(memory_space=pltpu.VMEM),
        ],
        out_specs=pl.BlockSpec(memory_space=pltpu.VMEM),
        scratch_shapes=[
            pltpu.VMEM((4, half, k_dim), jnp.float32),
            pltpu.VMEM((4, half, k_dim), jnp.float32),
            pltpu.VMEM((3, half, k_dim), jnp.float32),
            pltpu.VMEM((3, half, k_dim), jnp.float32),
            pltpu.SemaphoreType.DMA((6,)),
            pltpu.SemaphoreType.DMA((6,)),
            pltpu.SemaphoreType.DMA((6,)),
            pltpu.SemaphoreType.DMA((6,)),
            pltpu.SemaphoreType.DMA((4,)),
            pltpu.SemaphoreType.DMA((4,)),
            pltpu.SemaphoreType.DMA((3,)),
            pltpu.SemaphoreType.DMA((3,)),
            pltpu.SemaphoreType.DMA((3,)),
            pltpu.SemaphoreType.DMA((3,)),
            pltpu.SemaphoreType.REGULAR,
            pltpu.SemaphoreType.REGULAR,
            pltpu.SemaphoreType.REGULAR,
        ],
        compiler_params=pltpu.CompilerParams(collective_id=0),
    )(x, w_mat)
